# baseline (speedup 1.0000x reference)
import functools
import jax
import jax.numpy as jnp
from jax import lax
from jax.experimental import pallas as pl
from jax.experimental.pallas import tpu as pltpu
from jax.experimental.pallas import tpu_sc as plsc

N = 10000
E = 100000
IN_DIM = 128
HID = 128
OUT_DIM = 64
HEADS = 16
HDIM = 8
LAYERS = 4
RELS = 3

NP = 10112
NW = 32
CHUNK = 80
NCHUNK = 42
EW = CHUNK * NCHUNK
EPAD = EW * NW
RPT = NP // 16
TW = HEADS * HDIM + HEADS

ROWB = 1000
GRID = N // ROWB


def _mlp_body(x_ref, w1_ref, b1_ref, w2_ref, b2_ref, o_ref):
    h1 = jnp.dot(x_ref[...], w1_ref[...], preferred_element_type=jnp.float32)
    h1 = jnp.maximum(h1 + b1_ref[...], 0.0)
    o_ref[...] = jnp.dot(h1, w2_ref[...], preferred_element_type=jnp.float32) + b2_ref[...]


def _mlp(x, w1, b1, w2, b2, dout):
    return pl.pallas_call(
        _mlp_body,
        grid=(GRID,),
        in_specs=[
            pl.BlockSpec((ROWB, x.shape[1]), lambda i: (i, 0)),
            pl.BlockSpec(w1.shape, lambda i: (0, 0)),
            pl.BlockSpec((1, w1.shape[1]), lambda i: (0, 0)),
            pl.BlockSpec(w2.shape, lambda i: (0, 0)),
            pl.BlockSpec((1, dout), lambda i: (0, 0)),
        ],
        out_specs=pl.BlockSpec((ROWB, dout), lambda i: (i, 0)),
        out_shape=jax.ShapeDtypeStruct((N, dout), jnp.float32),
    )(x, w1, b1.reshape(1, -1), w2, b2.reshape(1, -1))


def _project_body(h_ref, wd_ref, ald_ref, ard_ref,
                  tbl_ref, er_ref, c_ref, mx_ref):
    i = pl.program_id(1)
    featd = jnp.dot(h_ref[...], wd_ref[0], preferred_element_type=jnp.float32)
    el = jnp.dot(featd, ald_ref[0], preferred_element_type=jnp.float32)
    er = jnp.dot(featd, ard_ref[0], preferred_element_type=jnp.float32)
    tbl_ref[0, :, 0:HEADS * HDIM] = featd
    tbl_ref[0, :, HEADS * HDIM:TW] = el
    er_ref[0] = er
    ml = jnp.max(el, axis=0, keepdims=True)
    mr = jnp.max(er, axis=0, keepdims=True)
    m = jnp.concatenate([ml, mr], axis=0)

    @pl.when(i == 0)
    def _():
        mx_ref[0] = m

    @pl.when(i != 0)
    def _():
        mx_ref[0] = jnp.maximum(mx_ref[0], m)

    @pl.when(i == GRID - 1)
    def _():
        mm = jnp.maximum(mx_ref[0], m)
        z = mm[0:1, :] + mm[1:2, :]
        c_ref[0] = jnp.where(z > 0, z, 0.2 * z)


def _project(h, wd, ald, ard):
    return pl.pallas_call(
        _project_body,
        grid=(RELS, GRID),
        in_specs=[
            pl.BlockSpec((ROWB, HID), lambda r, i: (i, 0)),
            pl.BlockSpec((1, HID, HID), lambda r, i: (r, 0, 0)),
            pl.BlockSpec((1, HID, HEADS), lambda r, i: (r, 0, 0)),
            pl.BlockSpec((1, HID, HEADS), lambda r, i: (r, 0, 0)),
        ],
        out_specs=[
            pl.BlockSpec((1, ROWB, TW), lambda r, i: (r, i, 0)),
            pl.BlockSpec((1, ROWB, HEADS), lambda r, i: (r, i, 0)),
            pl.BlockSpec((1, 1, HEADS), lambda r, i: (r, 0, 0)),
            pl.BlockSpec((1, 2, HEADS), lambda r, i: (r, 0, 0)),
        ],
        out_shape=[
            jax.ShapeDtypeStruct((RELS, N, TW), jnp.float32),
            jax.ShapeDtypeStruct((RELS, N, HEADS), jnp.float32),
            jax.ShapeDtypeStruct((RELS, 1, HEADS), jnp.float32),
            jax.ShapeDtypeStruct((RELS, 2, HEADS), jnp.float32),
        ],
    )(h, wd, ald, ard)[:3]


def _merge_body(parts_ref, h_ref, p_ref, b3_ref, o_ref):
    acc = None
    for r in range(RELS):
        s = parts_ref[r, 0] + parts_ref[r, 1]
        den = s[:, HEADS * HDIM:TW]
        dent = jnp.concatenate([den] * HDIM, axis=1)
        val = jnp.where(dent > 0, s[:, 0:HEADS * HDIM] / dent, 0.0)
        acc = val if acc is None else acc + val
    out = jnp.dot(acc, p_ref[...], preferred_element_type=jnp.float32)
    out = out + jnp.sum(b3_ref[...], axis=0, keepdims=True)
    o_ref[...] = jnp.where(out > 0, out, 0.01 * out) + h_ref[...]


def _merge(parts, h, perm, b3):
    return pl.pallas_call(
        _merge_body,
        grid=(GRID,),
        in_specs=[
            pl.BlockSpec((RELS, 2, ROWB, TW), lambda i: (0, 0, i, 0)),
            pl.BlockSpec((ROWB, HID), lambda i: (i, 0)),
            pl.BlockSpec((HID, HID), lambda i: (0, 0)),
            pl.BlockSpec((RELS, HID), lambda i: (0, 0)),
        ],
        out_specs=pl.BlockSpec((ROWB, HID), lambda i: (i, 0)),
        out_shape=jax.ShapeDtypeStruct((N, HID), jnp.float32),
    )(parts, h, perm, b3)


@functools.lru_cache(maxsize=1)
def _get_edge_kernel():
  mesh = plsc.VectorSubcoreMesh(core_axis_name="c", subcore_axis_name="s")

  @functools.partial(
    pl.kernel,
    out_type=jax.ShapeDtypeStruct((RELS, 2 * NP, TW), jnp.float32),
    mesh=mesh,
    scratch_types=dict(
        srcs=[pltpu.VMEM((CHUNK,), jnp.int32) for _ in range(3)],
        dsts=[pltpu.VMEM((CHUNK,), jnp.int32) for _ in range(3)],
        rows=[pltpu.VMEM((CHUNK, TW), jnp.float32) for _ in range(3)],
        errs=[pltpu.VMEM((CHUNK, HEADS), jnp.float32) for _ in range(3)],
        c_v=pltpu.VMEM((HEADS,), jnp.float32),
        accum=pltpu.VMEM_SHARED((NP, TW), jnp.float32),
        semg=[pltpu.SemaphoreType.DMA for _ in range(3)],
        sems=[pltpu.SemaphoreType.DMA for _ in range(3)],
    ),
    compiler_params=pltpu.CompilerParams(use_tc_tiling_on_sc=False),
  )
  def _edge_kernel(tbl_hbm, er_hbm, e0_hbm, e1_hbm, e2_hbm, c_hbm, parts_hbm,
                   srcs, dsts, rows, errs, c_v, accum, semg, sems):
    cid = lax.axis_index("c")
    sid = lax.axis_index("s")
    wid = sid * 2 + cid
    ebase = wid * EW
    rbase = sid * RPT
    zeros16 = jnp.zeros((16,), jnp.float32)

    def zrow(i, carry):
        for k in range(TW // 16):
            rows[0][i, pl.ds(k * 16, 16)] = zeros16
        return carry

    for r in range(RELS):
        ei = (e0_hbm, e1_hbm, e2_hbm)[r]
        tbl_r = tbl_hbm.at[r]
        er_r = er_hbm.at[r]
        pltpu.sync_copy(c_hbm.at[r, 0], c_v)
        cvec = c_v[...]
        lax.fori_loop(0, CHUNK, zrow, 0)
        for j in range(7):
            pltpu.sync_copy(rows[0], accum.at[pl.ds(rbase + j * CHUNK, CHUNK)])
        pltpu.sync_copy(rows[0].at[pl.ds(0, RPT - 7 * CHUNK)],
                        accum.at[pl.ds(rbase + 7 * CHUNK, RPT - 7 * CHUNK)])
        plsc.subcore_barrier()

        def issue_gather(c, x):
            base = ebase + c * CHUNK
            pltpu.sync_copy(ei.at[0, pl.ds(base, CHUNK)], srcs[x])
            pltpu.sync_copy(ei.at[1, pl.ds(base, CHUNK)], dsts[x])
            pltpu.async_copy(tbl_r.at[srcs[x]], rows[x], semg[x])
            pltpu.async_copy(er_r.at[dsts[x]], errs[x], semg[x])

        def wait_gather(x):
            pltpu.make_async_copy(tbl_r.at[srcs[x]], rows[x], semg[x]).wait()
            pltpu.make_async_copy(er_r.at[dsts[x]], errs[x], semg[x]).wait()

        def wait_scatter(x):
            pltpu.make_async_copy(rows[x], accum.at[dsts[x]], sems[x]).wait()

        def phase(c, x):
            wait_gather(x)
            rv = rows[x]
            ev = errs[x]

            @plsc.parallel_loop(0, CHUNK, unroll=4)
            def _(i):
                elv = rv[i, pl.ds(HEADS * HDIM, HEADS)]
                z = elv + ev[i, :]
                e = jnp.where(z > 0, z, 0.2 * z)
                ex = jnp.exp(e - cvec)
                rv[i, pl.ds(HEADS * HDIM, HEADS)] = ex
                for k in range(HDIM):
                    rv[i, pl.ds(k * 16, 16)] = rv[i, pl.ds(k * 16, 16)] * ex

            pltpu.async_copy(rv, accum.at[dsts[x]], sems[x], add=True)
            y = (x + 2) % 3

            @pl.when(c >= 1)
            def _():
                wait_scatter(y)

            @pl.when(c + 2 < NCHUNK)
            def _():
                issue_gather(c + 2, y)

        issue_gather(0, 0)
        issue_gather(1, 1)

        def ring_body(it, carry):
            c0 = it * 3
            phase(c0, 0)
            phase(c0 + 1, 1)
            phase(c0 + 2, 2)
            return carry

        lax.fori_loop(0, NCHUNK // 3, ring_body, 0)
        wait_scatter((NCHUNK - 1) % 3)
        plsc.subcore_barrier()

        obase = cid * NP + rbase
        for j in range(7):
            pltpu.sync_copy(accum.at[pl.ds(rbase + j * CHUNK, CHUNK)],
                            parts_hbm.at[r, pl.ds(obase + j * CHUNK, CHUNK)])
        pltpu.sync_copy(accum.at[pl.ds(rbase + 7 * CHUNK, RPT - 7 * CHUNK)],
                        parts_hbm.at[r, pl.ds(obase + 7 * CHUNK, RPT - 7 * CHUNK)])
        plsc.subcore_barrier()

  return _edge_kernel


def _prep_weights(params):
    d = jnp.arange(HID)
    perm = jnp.zeros((HID, HID), jnp.float32).at[d, (d % HEADS) * HDIM + d // HEADS].set(1.0)
    col = jnp.tile(jnp.arange(HEADS), HDIM)
    wd, ald, ard, b3 = [], [], [], []
    for layer in params['gnn']:
        wd.append(jnp.stack([
            p['W'].reshape(HID, HEADS, HDIM).transpose(0, 2, 1).reshape(HID, HID)
            for p in layer]))
        ald.append(jnp.stack([
            jnp.zeros((HID, HEADS), jnp.float32).at[d, col].set(p['al'].T.reshape(-1))
            for p in layer]))
        ard.append(jnp.stack([
            jnp.zeros((HID, HEADS), jnp.float32).at[d, col].set(p['ar'].T.reshape(-1))
            for p in layer]))
        b3.append(jnp.stack([p['b'] for p in layer]))
    return perm, wd, ald, ard, b3


def kernel(x, edge_index_0, edge_index_1, edge_index_2, params):
    pad = jnp.concatenate(
        [jnp.zeros((1, EPAD - E), jnp.int32),
         jnp.full((1, EPAD - E), N, jnp.int32)], axis=0)
    e0 = jnp.concatenate([edge_index_0, pad], axis=1)
    e1 = jnp.concatenate([edge_index_1, pad], axis=1)
    e2 = jnp.concatenate([edge_index_2, pad], axis=1)

    perm, wd, ald, ard, b3 = _prep_weights(params)

    emb = params['embed']
    h = _mlp(x, emb['W1'], emb['b1'], emb['W2'], emb['b2'], HID)

    for l in range(LAYERS):
        tbl, er, c = _project(h, wd[l], ald[l], ard[l])
        er_p = jnp.pad(er, ((0, 0), (0, NP - N), (0, 0)))
        parts = _get_edge_kernel()(tbl, er_p, e0, e1, e2, c)
        h = _merge(parts.reshape(RELS, 2, NP, TW), h, perm, b3[l])

    dec = params['dec']
    return _mlp(h, dec['W1'], dec['b1'], dec['W2'], dec['b2'], OUT_DIM)


# speedup vs baseline: 1.3820x; 1.3820x over previous
import functools
import jax
import jax.numpy as jnp
from jax import lax
from jax.experimental import pallas as pl
from jax.experimental.pallas import tpu as pltpu
from jax.experimental.pallas import tpu_sc as plsc

N = 10000
E = 100000
IN_DIM = 128
HID = 128
OUT_DIM = 64
HEADS = 16
HDIM = 8
LAYERS = 4
RELS = 3

NP = 10112
NW = 32
CHUNK = 64
NCHUNK = 51
EW = CHUNK * NCHUNK
EPAD = EW * NW
RPT = NP // 16
TW = HEADS * HDIM + HEADS

ROWB = 1000
GRID = N // ROWB


def _mlp_body(x_ref, w1_ref, b1_ref, w2_ref, b2_ref, o_ref):
    h1 = jnp.dot(x_ref[...], w1_ref[...], preferred_element_type=jnp.float32)
    h1 = jnp.maximum(h1 + b1_ref[...], 0.0)
    o_ref[...] = jnp.dot(h1, w2_ref[...], preferred_element_type=jnp.float32) + b2_ref[...]


def _mlp(x, w1, b1, w2, b2, dout):
    return pl.pallas_call(
        _mlp_body,
        grid=(GRID,),
        in_specs=[
            pl.BlockSpec((ROWB, x.shape[1]), lambda i: (i, 0)),
            pl.BlockSpec(w1.shape, lambda i: (0, 0)),
            pl.BlockSpec((1, w1.shape[1]), lambda i: (0, 0)),
            pl.BlockSpec(w2.shape, lambda i: (0, 0)),
            pl.BlockSpec((1, dout), lambda i: (0, 0)),
        ],
        out_specs=pl.BlockSpec((ROWB, dout), lambda i: (i, 0)),
        out_shape=jax.ShapeDtypeStruct((N, dout), jnp.float32),
    )(x, w1, b1.reshape(1, -1), w2, b2.reshape(1, -1))


def _project_body(h_ref, wd_ref, ald_ref, ard_ref,
                  tbl_ref, er_ref, c_ref, mx_ref):
    i = pl.program_id(1)
    featd = jnp.dot(h_ref[...], wd_ref[0], preferred_element_type=jnp.float32)
    el = jnp.dot(featd, ald_ref[0], preferred_element_type=jnp.float32)
    er = jnp.dot(featd, ard_ref[0], preferred_element_type=jnp.float32)
    tbl_ref[0, :, 0:HEADS * HDIM] = featd
    tbl_ref[0, :, HEADS * HDIM:TW] = el
    er_ref[0] = er
    ml = jnp.max(el, axis=0, keepdims=True)
    mr = jnp.max(er, axis=0, keepdims=True)
    m = jnp.concatenate([ml, mr], axis=0)

    @pl.when(i == 0)
    def _():
        mx_ref[0] = m

    @pl.when(i != 0)
    def _():
        mx_ref[0] = jnp.maximum(mx_ref[0], m)

    @pl.when(i == GRID - 1)
    def _():
        mm = jnp.maximum(mx_ref[0], m)
        z = mm[0:1, :] + mm[1:2, :]
        c_ref[0] = jnp.where(z > 0, z, 0.2 * z)


def _project(h, wd, ald, ard):
    return pl.pallas_call(
        _project_body,
        grid=(RELS, GRID),
        in_specs=[
            pl.BlockSpec((ROWB, HID), lambda r, i: (i, 0)),
            pl.BlockSpec((1, HID, HID), lambda r, i: (r, 0, 0)),
            pl.BlockSpec((1, HID, HEADS), lambda r, i: (r, 0, 0)),
            pl.BlockSpec((1, HID, HEADS), lambda r, i: (r, 0, 0)),
        ],
        out_specs=[
            pl.BlockSpec((1, ROWB, TW), lambda r, i: (r, i, 0)),
            pl.BlockSpec((1, ROWB, HEADS), lambda r, i: (r, i, 0)),
            pl.BlockSpec((1, 1, HEADS), lambda r, i: (r, 0, 0)),
            pl.BlockSpec((1, 2, HEADS), lambda r, i: (r, 0, 0)),
        ],
        out_shape=[
            jax.ShapeDtypeStruct((RELS, N, TW), jnp.float32),
            jax.ShapeDtypeStruct((RELS, N, HEADS), jnp.float32),
            jax.ShapeDtypeStruct((RELS, 1, HEADS), jnp.float32),
            jax.ShapeDtypeStruct((RELS, 2, HEADS), jnp.float32),
        ],
    )(h, wd, ald, ard)[:3]


def _merge_body(parts_ref, h_ref, p_ref, b3_ref, o_ref):
    acc = None
    for r in range(RELS):
        s = parts_ref[r, 0] + parts_ref[r, 1]
        den = s[:, HEADS * HDIM:TW]
        dent = jnp.concatenate([den] * HDIM, axis=1)
        val = jnp.where(dent > 0, s[:, 0:HEADS * HDIM] / dent, 0.0)
        acc = val if acc is None else acc + val
    out = jnp.dot(acc, p_ref[...], preferred_element_type=jnp.float32)
    out = out + jnp.sum(b3_ref[...], axis=0, keepdims=True)
    o_ref[...] = jnp.where(out > 0, out, 0.01 * out) + h_ref[...]


def _merge(parts, h, perm, b3):
    return pl.pallas_call(
        _merge_body,
        grid=(GRID,),
        in_specs=[
            pl.BlockSpec((RELS, 2, ROWB, TW), lambda i: (0, 0, i, 0)),
            pl.BlockSpec((ROWB, HID), lambda i: (i, 0)),
            pl.BlockSpec((HID, HID), lambda i: (0, 0)),
            pl.BlockSpec((RELS, HID), lambda i: (0, 0)),
        ],
        out_specs=pl.BlockSpec((ROWB, HID), lambda i: (i, 0)),
        out_shape=jax.ShapeDtypeStruct((N, HID), jnp.float32),
    )(parts, h, perm, b3)


@functools.lru_cache(maxsize=1)
def _get_edge_kernel():
  mesh = plsc.VectorSubcoreMesh(core_axis_name="c", subcore_axis_name="s")

  @functools.partial(
    pl.kernel,
    out_type=jax.ShapeDtypeStruct((RELS, 2 * NP, TW), jnp.float32),
    mesh=mesh,
    scratch_types=dict(
        src_all=pltpu.VMEM((NCHUNK, CHUNK), jnp.int32),
        dst_all=pltpu.VMEM((NCHUNK, CHUNK), jnp.int32),
        rows=[pltpu.VMEM((CHUNK, TW), jnp.float32) for _ in range(3)],
        errs=[pltpu.VMEM((CHUNK, HEADS), jnp.float32) for _ in range(3)],
        c_v=pltpu.VMEM((HEADS,), jnp.float32),
        accum=pltpu.VMEM_SHARED((NP, TW), jnp.float32),
        semg=[pltpu.SemaphoreType.DMA for _ in range(3)],
        sems=[pltpu.SemaphoreType.DMA for _ in range(3)],
    ),
    compiler_params=pltpu.CompilerParams(use_tc_tiling_on_sc=False),
  )
  def _edge_kernel(tbl_hbm, er_hbm, e0_hbm, e1_hbm, e2_hbm, c_hbm, parts_hbm,
                   src_all, dst_all, rows, errs, c_v, accum, semg, sems):
    cid = lax.axis_index("c")
    sid = lax.axis_index("s")
    wid = sid * 2 + cid
    rbase = sid * RPT
    zeros16 = jnp.zeros((16,), jnp.float32)

    def zrow(i, carry):
        for k in range(TW // 16):
            rows[0][i, pl.ds(k * 16, 16)] = zeros16
        return carry

    for r in range(RELS):
        ei = (e0_hbm, e1_hbm, e2_hbm)[r]
        tbl_r = tbl_hbm.at[r]
        er_r = er_hbm.at[r]
        pltpu.sync_copy(c_hbm.at[r, 0], c_v)
        cvec = c_v[...]
        pltpu.sync_copy(ei.at[0, wid], src_all)
        pltpu.sync_copy(ei.at[1, wid], dst_all)
        lax.fori_loop(0, CHUNK, zrow, 0)
        for j in range(RPT // CHUNK):
            pltpu.sync_copy(rows[0], accum.at[pl.ds(rbase + j * CHUNK, CHUNK)])
        rem = RPT - (RPT // CHUNK) * CHUNK
        if rem:
            pltpu.sync_copy(rows[0].at[pl.ds(0, rem)],
                            accum.at[pl.ds(rbase + RPT - rem, rem)])
        plsc.subcore_barrier()

        def issue_gather(c, x):
            pltpu.async_copy(tbl_r.at[src_all.at[c]], rows[x], semg[x])
            pltpu.async_copy(er_r.at[dst_all.at[c]], errs[x], semg[x])

        def wait_gather(c, x):
            pltpu.make_async_copy(tbl_r.at[src_all.at[c]], rows[x], semg[x]).wait()
            pltpu.make_async_copy(er_r.at[dst_all.at[c]], errs[x], semg[x]).wait()

        def wait_scatter(c, x):
            pltpu.make_async_copy(rows[x], accum.at[dst_all.at[c]], sems[x]).wait()

        def phase(c, x):
            wait_gather(c, x)
            rv = rows[x]
            ev = errs[x]

            @plsc.parallel_loop(0, CHUNK, unroll=4)
            def _(i):
                elv = rv[i, pl.ds(HEADS * HDIM, HEADS)]
                z = elv + ev[i, :]
                e = jnp.where(z > 0, z, 0.2 * z)
                ex = jnp.exp(e - cvec)
                rv[i, pl.ds(HEADS * HDIM, HEADS)] = ex
                for k in range(HDIM):
                    rv[i, pl.ds(k * 16, 16)] = rv[i, pl.ds(k * 16, 16)] * ex

            pltpu.async_copy(rv, accum.at[dst_all.at[c]], sems[x], add=True)
            y = (x + 2) % 3

            @pl.when(c >= 1)
            def _():
                wait_scatter(c - 1, y)

            @pl.when(c + 2 < NCHUNK)
            def _():
                issue_gather(c + 2, y)

        issue_gather(0, 0)
        issue_gather(1, 1)

        def ring_body(it, carry):
            c0 = it * 3
            phase(c0, 0)
            phase(c0 + 1, 1)
            phase(c0 + 2, 2)
            return carry

        lax.fori_loop(0, NCHUNK // 3, ring_body, 0)
        wait_scatter(NCHUNK - 1, (NCHUNK - 1) % 3)
        plsc.subcore_barrier()

        obase = cid * NP + rbase
        pltpu.sync_copy(accum.at[pl.ds(rbase, RPT)],
                        parts_hbm.at[r, pl.ds(obase, RPT)])
        plsc.subcore_barrier()

  return _edge_kernel


def _prep_weights(params):
    d = jnp.arange(HID)
    perm = jnp.zeros((HID, HID), jnp.float32).at[d, (d % HEADS) * HDIM + d // HEADS].set(1.0)
    col = jnp.tile(jnp.arange(HEADS), HDIM)
    wd, ald, ard, b3 = [], [], [], []
    for layer in params['gnn']:
        wd.append(jnp.stack([
            p['W'].reshape(HID, HEADS, HDIM).transpose(0, 2, 1).reshape(HID, HID)
            for p in layer]))
        ald.append(jnp.stack([
            jnp.zeros((HID, HEADS), jnp.float32).at[d, col].set(p['al'].T.reshape(-1))
            for p in layer]))
        ard.append(jnp.stack([
            jnp.zeros((HID, HEADS), jnp.float32).at[d, col].set(p['ar'].T.reshape(-1))
            for p in layer]))
        b3.append(jnp.stack([p['b'] for p in layer]))
    return perm, wd, ald, ard, b3


def kernel(x, edge_index_0, edge_index_1, edge_index_2, params):
    pad = jnp.concatenate(
        [jnp.zeros((1, EPAD - E), jnp.int32),
         jnp.full((1, EPAD - E), N, jnp.int32)], axis=0)
    e0 = jnp.concatenate([edge_index_0, pad], axis=1).reshape(2, NW, NCHUNK, CHUNK)
    e1 = jnp.concatenate([edge_index_1, pad], axis=1).reshape(2, NW, NCHUNK, CHUNK)
    e2 = jnp.concatenate([edge_index_2, pad], axis=1).reshape(2, NW, NCHUNK, CHUNK)

    perm, wd, ald, ard, b3 = _prep_weights(params)

    emb = params['embed']
    h = _mlp(x, emb['W1'], emb['b1'], emb['W2'], emb['b2'], HID)

    for l in range(LAYERS):
        tbl, er, c = _project(h, wd[l], ald[l], ard[l])
        er_p = jnp.pad(er, ((0, 0), (0, NP - N), (0, 0)))
        parts = _get_edge_kernel()(tbl, er_p, e0, e1, e2, c)
        h = _merge(parts.reshape(RELS, 2, NP, TW), h, perm, b3[l])

    dec = params['dec']
    return _mlp(h, dec['W1'], dec['b1'], dec['W2'], dec['b2'], OUT_DIM)
